# baseline (speedup 1.0000x reference)
import functools
import math

import jax
import jax.numpy as jnp
from jax import lax
from jax.experimental import pallas as pl
from jax.experimental.pallas import tpu as pltpu
from jax.experimental.pallas import tpu_sc as plsc

N = 10000
DEG = 16
D = 256
TPERC = 0.45
REMOVE = math.floor(DEG * TPERC)
LO = REMOVE
HI = DEG - REMOVE - 1

NC = 2
NS = 16
L = 16
NW = NC * NS

CH = 8
ROWS = CH * DEG
NCHUNKS = N // CH
KMAX = -(-NCHUNKS // NW)


def _mm_body(x_ref, wt_ref, o_ref):
    o_ref[...] = jnp.dot(x_ref[...], wt_ref[...],
                         preferred_element_type=jnp.float32)


def _project(x, wt):
    m = x.shape[0]
    blk = 1000
    grid = m // blk
    return pl.pallas_call(
        _mm_body,
        grid=(grid,),
        in_specs=[
            pl.BlockSpec((blk, D), lambda i: (i, 0)),
            pl.BlockSpec((D, D), lambda i: (0, 0)),
        ],
        out_specs=pl.BlockSpec((blk, D), lambda i: (i, 0)),
        out_shape=jax.ShapeDtypeStruct((m, D), jnp.float32),
    )(x, wt)


_SORT8 = ((0, 1), (2, 3), (4, 5), (6, 7), (0, 2), (1, 3), (4, 6), (5, 7),
          (1, 2), (5, 6), (0, 4), (1, 5), (2, 6), (3, 7), (2, 4), (3, 5),
          (1, 2), (3, 4), (5, 6))


def _median16(vs):
    vs = list(vs)
    for off in (0, 8):
        for (i, j) in _SORT8:
            a, b = vs[off + i], vs[off + j]
            vs[off + i] = jnp.minimum(a, b)
            vs[off + j] = jnp.maximum(a, b)
    lo = [jnp.minimum(vs[i], vs[15 - i]) for i in range(8)]
    hi = [jnp.maximum(vs[i], vs[15 - i]) for i in range(8)]
    while len(lo) > 1:
        lo = [jnp.maximum(lo[k], lo[k + 1]) for k in range(0, len(lo), 2)]
        hi = [jnp.minimum(hi[k], hi[k + 1]) for k in range(0, len(hi), 2)]
    return (lo[0] + hi[0]) * 0.5


_sc_mesh = plsc.VectorSubcoreMesh(core_axis_name="c", subcore_axis_name="s")


@functools.partial(
    pl.kernel,
    mesh=_sc_mesh,
    out_type=jax.ShapeDtypeStruct((N, D), jnp.float32),
    scratch_types=[
        pltpu.VMEM((ROWS,), jnp.int32),
        pltpu.VMEM((ROWS,), jnp.int32),
        pltpu.VMEM((ROWS, D), jnp.float32),
        pltpu.VMEM((ROWS, D), jnp.float32),
        pltpu.VMEM((CH, D), jnp.float32),
        pltpu.SemaphoreType.DMA,
        pltpu.SemaphoreType.DMA,
    ],
    compiler_params=pltpu.CompilerParams(
        use_tc_tiling_on_sc=False, needs_layout_passes=False
    ),
)
def _sc_median(h_hbm, nbrs_hbm, out_hbm, idx0, idx1, rows0, rows1, out_v,
               sem0, sem1):
    cid = lax.axis_index("c")
    sid = lax.axis_index("s")
    wid = cid * NS + sid
    idx_b = (idx0, idx1)
    rows_b = (rows0, rows1)
    sem_b = (sem0, sem1)

    def start(k, b):
        chunk = wid + NW * k

        @pl.when(chunk < NCHUNKS)
        def _():
            pltpu.sync_copy(nbrs_hbm.at[pl.ds(chunk * ROWS, ROWS)], idx_b[b])
            pltpu.async_copy(h_hbm.at[idx_b[b]], rows_b[b], sem_b[b])

    def finish(k, b):
        chunk = wid + NW * k

        @pl.when(chunk < NCHUNKS)
        def _():
            pltpu.make_async_copy(h_hbm.at[idx_b[b]], rows_b[b],
                                  sem_b[b]).wait()
            rows_v = rows_b[b]
            for n in range(CH):

                def grp_body(gg, carry2):
                    cs = pl.ds(gg * L, L)
                    vs = [rows_v[n * DEG + j, cs] for j in range(DEG)]
                    out_v[n, cs] = _median16(vs)
                    return carry2

                lax.fori_loop(0, D // L, grp_body, 0, unroll=4)
            pltpu.sync_copy(out_v, out_hbm.at[pl.ds(chunk * CH, CH)])

    start(0, 0)

    def pair_body(k2, carry):
        k = 2 * k2
        start(k + 1, 1)
        finish(k, 0)
        start(k + 2, 0)
        finish(k + 1, 1)
        return carry

    lax.fori_loop(0, KMAX // 2, pair_body, 0)


def kernel(x, nbrs, W):
    h = _project(x, W.T)
    return _sc_median(h, nbrs.reshape(-1))


# speedup vs baseline: 1.4004x; 1.4004x over previous
import functools
import math

import jax
import jax.numpy as jnp
from jax import lax
from jax.experimental import pallas as pl
from jax.experimental.pallas import tpu as pltpu
from jax.experimental.pallas import tpu_sc as plsc

N = 10000
DEG = 16
D = 256
TPERC = 0.45
REMOVE = math.floor(DEG * TPERC)
LO = REMOVE
HI = DEG - REMOVE - 1

NC = 2
NS = 16
L = 16
NW = NC * NS

CH = 8
ROWS = CH * DEG
NCHUNKS = N // CH
KMAX = -(-NCHUNKS // NW)


def _mm_body(x_ref, wt_ref, o_ref):
    o_ref[...] = jnp.dot(x_ref[...], wt_ref[...],
                         preferred_element_type=jnp.float32)


def _project(x, wt):
    m = x.shape[0]
    blk = 1000
    grid = m // blk
    return pl.pallas_call(
        _mm_body,
        grid=(grid,),
        in_specs=[
            pl.BlockSpec((blk, D), lambda i: (i, 0)),
            pl.BlockSpec((D, D), lambda i: (0, 0)),
        ],
        out_specs=pl.BlockSpec((blk, D), lambda i: (i, 0)),
        out_shape=jax.ShapeDtypeStruct((m, D), jnp.float32),
    )(x, wt)


_SORT8 = ((0, 1), (2, 3), (4, 5), (6, 7), (0, 2), (1, 3), (4, 6), (5, 7),
          (1, 2), (5, 6), (0, 4), (1, 5), (2, 6), (3, 7), (2, 4), (3, 5),
          (1, 2), (3, 4), (5, 6))


def _median16(vs):
    vs = list(vs)
    for off in (0, 8):
        for (i, j) in _SORT8:
            a, b = vs[off + i], vs[off + j]
            vs[off + i] = jnp.minimum(a, b)
            vs[off + j] = jnp.maximum(a, b)
    lo = [jnp.minimum(vs[i], vs[15 - i]) for i in range(8)]
    hi = [jnp.maximum(vs[i], vs[15 - i]) for i in range(8)]
    while len(lo) > 1:
        lo = [jnp.maximum(lo[k], lo[k + 1]) for k in range(0, len(lo), 2)]
        hi = [jnp.minimum(hi[k], hi[k + 1]) for k in range(0, len(hi), 2)]
    return (lo[0] + hi[0]) * 0.5


_sc_mesh = plsc.VectorSubcoreMesh(core_axis_name="c", subcore_axis_name="s")


@functools.partial(
    pl.kernel,
    mesh=_sc_mesh,
    out_type=jax.ShapeDtypeStruct((N, D), jnp.float32),
    scratch_types=[
        pltpu.VMEM((ROWS,), jnp.int32),
        pltpu.VMEM((ROWS,), jnp.int32),
        pltpu.VMEM((ROWS, D), jnp.float32),
        pltpu.VMEM((ROWS, D), jnp.float32),
        pltpu.VMEM((CH, D), jnp.float32),
        pltpu.SemaphoreType.DMA,
        pltpu.SemaphoreType.DMA,
    ],
    compiler_params=pltpu.CompilerParams(
        use_tc_tiling_on_sc=False, needs_layout_passes=False
    ),
)
def _sc_median(h_hbm, nbrs_hbm, out_hbm, idx0, idx1, rows0, rows1, out_v,
               sem0, sem1):
    cid = lax.axis_index("c")
    sid = lax.axis_index("s")
    wid = cid * NS + sid
    idx_b = (idx0, idx1)
    rows_b = (rows0, rows1)
    sem_b = (sem0, sem1)

    def start(k, b):
        chunk = wid + NW * k

        @pl.when(chunk < NCHUNKS)
        def _():
            pltpu.sync_copy(nbrs_hbm.at[pl.ds(chunk * ROWS, ROWS)], idx_b[b])
            pltpu.async_copy(h_hbm.at[idx_b[b]], rows_b[b], sem_b[b])

    def finish(k, b):
        chunk = wid + NW * k

        @pl.when(chunk < NCHUNKS)
        def _():
            pltpu.make_async_copy(h_hbm.at[idx_b[b]], rows_b[b],
                                  sem_b[b]).wait()
            rows_v = rows_b[b]
            for n in range(CH):

                @plsc.parallel_loop(0, D // L, unroll=2)
                def grp_body(gg):
                    cs = pl.ds(gg * L, L)
                    vs = [rows_v[n * DEG + j, cs] for j in range(DEG)]
                    out_v[n, cs] = _median16(vs)
            pltpu.sync_copy(out_v, out_hbm.at[pl.ds(chunk * CH, CH)])

    start(0, 0)

    def pair_body(k2, carry):
        k = 2 * k2
        start(k + 1, 1)
        finish(k, 0)
        start(k + 2, 0)
        finish(k + 1, 1)
        return carry

    lax.fori_loop(0, KMAX // 2, pair_body, 0)


def kernel(x, nbrs, W):
    h = _project(x, W.T)
    return _sc_median(h, nbrs.reshape(-1))


# speedup vs baseline: 1.9163x; 1.3683x over previous
import functools
import math

import jax
import jax.numpy as jnp
from jax import lax
from jax.experimental import pallas as pl
from jax.experimental.pallas import tpu as pltpu
from jax.experimental.pallas import tpu_sc as plsc

N = 10000
DEG = 16
D = 256
TPERC = 0.45
REMOVE = math.floor(DEG * TPERC)
LO = REMOVE
HI = DEG - REMOVE - 1

NC = 2
NS = 16
L = 16
NW = NC * NS

CH = 8
ROWS = CH * DEG
NCHUNKS = N // CH
KMAX = -(-NCHUNKS // NW)


def _mm_body(x_ref, wt_ref, o_ref):
    o_ref[...] = jnp.dot(x_ref[...], wt_ref[...],
                         preferred_element_type=jnp.float32)


def _project(x, wt):
    m = x.shape[0]
    blk = 1000
    grid = m // blk
    return pl.pallas_call(
        _mm_body,
        grid=(grid,),
        in_specs=[
            pl.BlockSpec((blk, D), lambda i: (i, 0)),
            pl.BlockSpec((D, D), lambda i: (0, 0)),
        ],
        out_specs=pl.BlockSpec((blk, D), lambda i: (i, 0)),
        out_shape=jax.ShapeDtypeStruct((m, D), jnp.float32),
    )(x, wt)


_SORT8 = ((0, 1), (2, 3), (4, 5), (6, 7), (0, 2), (1, 3), (4, 6), (5, 7),
          (1, 2), (5, 6), (0, 4), (1, 5), (2, 6), (3, 7), (2, 4), (3, 5),
          (1, 2), (3, 4), (5, 6))


def _median16(vs):
    vs = list(vs)
    for off in (0, 8):
        for (i, j) in _SORT8:
            a, b = vs[off + i], vs[off + j]
            vs[off + i] = jnp.minimum(a, b)
            vs[off + j] = jnp.maximum(a, b)
    lo = [jnp.minimum(vs[i], vs[15 - i]) for i in range(8)]
    hi = [jnp.maximum(vs[i], vs[15 - i]) for i in range(8)]
    while len(lo) > 1:
        lo = [jnp.maximum(lo[k], lo[k + 1]) for k in range(0, len(lo), 2)]
        hi = [jnp.minimum(hi[k], hi[k + 1]) for k in range(0, len(hi), 2)]
    return (lo[0] + hi[0]) * 0.5


_sc_mesh = plsc.VectorSubcoreMesh(core_axis_name="c", subcore_axis_name="s")


@functools.partial(
    pl.kernel,
    mesh=_sc_mesh,
    out_type=jax.ShapeDtypeStruct((N, D), jnp.float32),
    scratch_types=[
        pltpu.VMEM((ROWS,), jnp.int32),
        pltpu.VMEM((ROWS,), jnp.int32),
        pltpu.VMEM((ROWS, D), jnp.float32),
        pltpu.VMEM((ROWS, D), jnp.float32),
        pltpu.VMEM((CH, D), jnp.float32),
        pltpu.SemaphoreType.DMA,
        pltpu.SemaphoreType.DMA,
    ],
    compiler_params=pltpu.CompilerParams(
        use_tc_tiling_on_sc=False, needs_layout_passes=False
    ),
)
def _sc_median(h_hbm, nbrs_hbm, out_hbm, idx0, idx1, rows0, rows1, out_v,
               sem0, sem1):
    cid = lax.axis_index("c")
    sid = lax.axis_index("s")
    wid = cid * NS + sid
    idx_b = (idx0, idx1)
    rows_b = (rows0, rows1)
    sem_b = (sem0, sem1)

    def start(k, b):
        chunk = wid + NW * k

        @pl.when(chunk < NCHUNKS)
        def _():
            pltpu.sync_copy(nbrs_hbm.at[pl.ds(chunk * ROWS, ROWS)], idx_b[b])
            pltpu.async_copy(h_hbm.at[idx_b[b]], rows_b[b], sem_b[b])

    def finish(k, b):
        chunk = wid + NW * k

        @pl.when(chunk < NCHUNKS)
        def _():
            pltpu.make_async_copy(h_hbm.at[idx_b[b]], rows_b[b],
                                  sem_b[b]).wait()
            rows_v = rows_b[b]

            @plsc.parallel_loop(0, CH * (D // L), unroll=2)
            def grp_body(i):
                n = i // (D // L)
                gg = lax.rem(i, D // L)
                cs = pl.ds(gg * L, L)
                vs = [rows_v[n * DEG + j, cs] for j in range(DEG)]
                out_v[n, cs] = _median16(vs)
            pltpu.sync_copy(out_v, out_hbm.at[pl.ds(chunk * CH, CH)])

    start(0, 0)

    def pair_body(k2, carry):
        k = 2 * k2
        start(k + 1, 1)
        finish(k, 0)
        start(k + 2, 0)
        finish(k + 1, 1)
        return carry

    lax.fori_loop(0, KMAX // 2, pair_body, 0)


def kernel(x, nbrs, W):
    h = _project(x, W.T)
    return _sc_median(h, nbrs.reshape(-1))
